# baseline (speedup 1.0000x reference)
import functools

import jax
import jax.numpy as jnp
from jax.experimental import pallas as pl

N_DIR = 9
BLK = 5000


def _body(x_ref, w_ref, b_ref, *out_refs):
    x = x_ref[...]
    for k in range(N_DIR):
        y = jax.lax.dot_general(
            x, w_ref[k],
            dimension_numbers=(((1,), (1,)), ((), ())),
            preferred_element_type=jnp.float32,
        )
        out_refs[k][...] = y + b_ref[k]


@functools.partial(jax.jit, static_argnums=())
def kernel(inputs, weights, bias):
    n, in_ch = inputs.shape
    _, out_ch, _ = weights.shape
    grid = (n // BLK,)
    out_shape = tuple(
        jax.ShapeDtypeStruct((n, out_ch), jnp.float32) for _ in range(N_DIR)
    )
    out_specs = tuple(
        pl.BlockSpec((BLK, out_ch), lambda i: (i, 0)) for _ in range(N_DIR)
    )
    outs = pl.pallas_call(
        _body,
        grid=grid,
        in_specs=[
            pl.BlockSpec((BLK, in_ch), lambda i: (i, 0)),
            pl.BlockSpec(weights.shape, lambda i: (0, 0, 0)),
            pl.BlockSpec(bias.shape, lambda i: (0, 0)),
        ],
        out_specs=out_specs,
        out_shape=out_shape,
    )(inputs, weights, bias)
    return outs


# speedup vs baseline: 1.0223x; 1.0223x over previous
import functools

import jax
import jax.numpy as jnp
from jax.experimental import pallas as pl
from jax.experimental.pallas import tpu as pltpu

N_DIR = 9
BLK = 4000


def _body(x_ref, w_ref, b_ref, *out_refs):
    x = x_ref[...]
    for k in range(N_DIR):
        y = jax.lax.dot_general(
            x, w_ref[k],
            dimension_numbers=(((1,), (1,)), ((), ())),
            preferred_element_type=jnp.float32,
        )
        out_refs[k][...] = y + b_ref[k]


@functools.partial(jax.jit, static_argnums=())
def kernel(inputs, weights, bias):
    n, in_ch = inputs.shape
    _, out_ch, _ = weights.shape
    grid = (n // BLK,)
    out_shape = tuple(
        jax.ShapeDtypeStruct((n, out_ch), jnp.float32) for _ in range(N_DIR)
    )
    out_specs = tuple(
        pl.BlockSpec((BLK, out_ch), lambda i: (i, 0)) for _ in range(N_DIR)
    )
    outs = pl.pallas_call(
        _body,
        grid=grid,
        in_specs=[
            pl.BlockSpec((BLK, in_ch), lambda i: (i, 0)),
            pl.BlockSpec(weights.shape, lambda i: (0, 0, 0)),
            pl.BlockSpec(bias.shape, lambda i: (0, 0)),
        ],
        out_specs=out_specs,
        out_shape=out_shape,
        compiler_params=pltpu.CompilerParams(dimension_semantics=("parallel",)),
    )(inputs, weights, bias)
    return outs
